# Initial kernel scaffold; baseline (speedup 1.0000x reference)
import jax
import jax.numpy as jnp
from jax.experimental import pallas as pl


def kernel(uEmbeds, iEmbeds, uHyper, iHyper, u_w, i_w, u_concat_w, i_concat_w, edge_vals, edge_index):
    raise NotImplementedError("write your pallas kernel here")


# speedup vs baseline: 3.6957x; 3.6957x over previous
import functools

import jax
import jax.numpy as jnp
from jax import lax
from jax.experimental import pallas as pl
from jax.experimental.pallas import tpu as pltpu
from jax.experimental.pallas import tpu_sc as plsc

USER = 3000
ITEM = 7000
N = USER + ITEM
D = 128
L = 2
B = 2
E = 320000
HOPS = 2

NC = 2
NS = 16
EPT = E // NS
C = 80
NCHUNK = EPT // C
SEG = 25
NSEG = NCHUNK // SEG
NPAD = 10240
RPT = NPAD // NS

def _spmm2_body(xall, src1, src2, dstr, valr, zeros, cur1, cur2,
                src_v, dst_v, val_v, rows_v, acc_s, sem):
    l = lax.axis_index("c")
    w = lax.axis_index("s")
    for b in range(B):
        for hop in range(HOPS):
            table = xall if hop == 0 else cur1
            srcarr = src1 if hop == 0 else src2
            out = cur1 if hop == 0 else cur2
            pltpu.sync_copy(zeros, acc_s.at[pl.ds(w * RPT, RPT)])
            plsc.subcore_barrier()

            def seg(s, _, table=table, srcarr=srcarr):
                pltpu.sync_copy(srcarr.at[l, b, w, s], src_v)
                pltpu.sync_copy(dstr.at[b, w, s], dst_v)
                pltpu.sync_copy(valr.at[b, w, s], val_v)

                def chunk(j, _):
                    pltpu.async_copy(table.at[src_v.at[j]], rows_v,
                                     sem).wait()

                    def scale16(g, _):
                        vv = val_v[j, pl.ds(g * 16, 16)]
                        for i in range(16):
                            cc = g * 16 + i
                            v = vv[i]
                            for k in range(D // 16):
                                sl = pl.ds(k * 16, 16)
                                rows_v[cc, sl] = rows_v[cc, sl] * v
                        return ()

                    lax.fori_loop(0, C // 16, scale16, ())
                    pltpu.sync_copy(rows_v, acc_s.at[dst_v.at[j]], add=True)
                    return ()

                lax.fori_loop(0, SEG, chunk, ())
                return ()

            lax.fori_loop(0, NSEG, seg, ())
            plsc.subcore_barrier()
            base = (b * L + l) * NPAD
            pltpu.sync_copy(acc_s.at[pl.ds(w * RPT, RPT)],
                            out.at[pl.ds(base + w * RPT, RPT)])
            plsc.subcore_barrier()


@functools.cache
def _get_spmm2():
    mesh = plsc.VectorSubcoreMesh(
        core_axis_name="c", subcore_axis_name="s",
        num_cores=NC, num_subcores=NS)
    return pl.kernel(
        _spmm2_body,
        out_type=[
            jax.ShapeDtypeStruct((B * L * NPAD, D), jnp.float32),
            jax.ShapeDtypeStruct((B * L * NPAD, D), jnp.float32),
        ],
        mesh=mesh,
        scratch_types=[
            pltpu.VMEM((SEG, C), jnp.int32),
            pltpu.VMEM((SEG, C), jnp.int32),
            pltpu.VMEM((SEG, C), jnp.float32),
            pltpu.VMEM((C, D), jnp.float32),
            pltpu.VMEM_SHARED((NPAD, D), jnp.float32),
            pltpu.SemaphoreType.DMA,
        ],
    )


BLK = 1000
NBLK = N // BLK
UBLK = USER // BLK


def _stats_body(x0_ref, c1_ref, c2_ref, acc_ref, su_ref, si_ref):
    r = pl.program_id(2)
    a = x0_ref[0] + c1_ref[0, 0] + c2_ref[0, 0]
    acc_ref[0, 0] = a
    p = lax.dot_general(a, a, (((0,), (0,)), ((), ())),
                        precision=lax.Precision.HIGHEST)

    @pl.when(r == 0)
    def _():
        su_ref[...] = jnp.zeros_like(su_ref)
        si_ref[...] = jnp.zeros_like(si_ref)

    is_user = (r < UBLK).astype(jnp.float32)
    su_ref[0, 0] += is_user * p
    si_ref[0, 0] += (1.0 - is_user) * p


_stats = pl.pallas_call(
    _stats_body,
    grid=(L, B, NBLK),
    in_specs=[
        pl.BlockSpec((1, BLK, D), lambda l, b, r: (l, r, 0)),
        pl.BlockSpec((1, 1, BLK, D), lambda l, b, r: (b, l, r, 0)),
        pl.BlockSpec((1, 1, BLK, D), lambda l, b, r: (b, l, r, 0)),
    ],
    out_specs=[
        pl.BlockSpec((1, 1, BLK, D), lambda l, b, r: (l, b, r, 0)),
        pl.BlockSpec((1, 1, D, D), lambda l, b, r: (l, b, 0, 0)),
        pl.BlockSpec((1, 1, D, D), lambda l, b, r: (l, b, 0, 0)),
    ],
    out_shape=[
        jax.ShapeDtypeStruct((L, B, N, D), jnp.float32),
        jax.ShapeDtypeStruct((L, B, D, D), jnp.float32),
        jax.ShapeDtypeStruct((L, B, D, D), jnp.float32),
    ],
)


def _mid_body(uh_ref, ih_ref, su_ref, si_ref, mu_ref, mi_ref):
    for l in range(L):
        gu = lax.dot_general(uh_ref[l], uh_ref[l], (((0,), (0,)), ((), ())),
                             precision=lax.Precision.HIGHEST)
        gi = lax.dot_general(ih_ref[l], ih_ref[l], (((0,), (0,)), ((), ())),
                             precision=lax.Precision.HIGHEST)
        for b in range(B):
            mu_ref[l, b] = jnp.dot(gu, su_ref[l, b],
                                   precision=lax.Precision.HIGHEST)
            mi_ref[l, b] = jnp.dot(gi, si_ref[l, b],
                                   precision=lax.Precision.HIGHEST)


_mid = pl.pallas_call(
    _mid_body,
    out_shape=[
        jax.ShapeDtypeStruct((L, B, D, D), jnp.float32),
        jax.ShapeDtypeStruct((L, B, D, D), jnp.float32),
    ],
)


def _emit_body(acc_ref, m_ref, w_ref, cw_ref, emb_ref, embs_ref):
    emb = jnp.zeros((BLK, D), jnp.float32)
    embs = [jnp.zeros((BLK, D), jnp.float32) for _ in range(B)]
    for l in range(L):
        ts = []
        for b in range(B):
            t_lb = jnp.dot(acc_ref[l, b], m_ref[l, b],
                           precision=lax.Precision.HIGHEST)
            ts.append(t_lb)
            embs[b] = embs[b] + jnp.dot(
                jax.nn.relu(jnp.dot(t_lb, w_ref[l],
                                    precision=lax.Precision.HIGHEST)),
                cw_ref[l], precision=lax.Precision.HIGHEST)
        tm = (ts[0] + ts[1]) * (1.0 / B)
        emb = emb + jnp.dot(
            jax.nn.relu(jnp.dot(tm, w_ref[l],
                                precision=lax.Precision.HIGHEST)),
            cw_ref[l], precision=lax.Precision.HIGHEST)
    emb_ref[...] = emb
    for b in range(B):
        embs_ref[b] = embs[b]


def _make_emit(rows, row_off_blocks):
    nblk = rows // BLK
    return pl.pallas_call(
        _emit_body,
        grid=(nblk,),
        in_specs=[
            pl.BlockSpec((L, B, BLK, D), lambda r: (0, 0, r + row_off_blocks, 0)),
            pl.BlockSpec((L, B, D, D), lambda r: (0, 0, 0, 0)),
            pl.BlockSpec((L, D, D), lambda r: (0, 0, 0)),
            pl.BlockSpec((L, D, D), lambda r: (0, 0, 0)),
        ],
        out_specs=[
            pl.BlockSpec((BLK, D), lambda r: (r, 0)),
            pl.BlockSpec((B, BLK, D), lambda r: (0, r, 0)),
        ],
        out_shape=[
            jax.ShapeDtypeStruct((rows, D), jnp.float32),
            jax.ShapeDtypeStruct((B, rows, D), jnp.float32),
        ],
    )


_emit_user = _make_emit(USER, 0)
_emit_item = _make_emit(ITEM, UBLK)


def kernel(uEmbeds, iEmbeds, uHyper, iHyper, u_w, i_w,
           u_concat_w, i_concat_w, edge_vals, edge_index):
    x0 = jnp.concatenate([uEmbeds, iEmbeds], axis=1)
    xall = x0.reshape(L * N, D)

    dst = edge_index[:, 0, :].reshape(B, NS, NSEG, SEG, C)
    src = edge_index[:, 1, :].reshape(B, NS, NSEG, SEG, C)
    valr = edge_vals.reshape(B, NS, NSEG, SEG, C)
    loff = (jnp.arange(L, dtype=jnp.int32) * N).reshape(L, 1, 1, 1, 1, 1)
    src1 = src[None] + loff
    base2 = ((jnp.arange(B, dtype=jnp.int32)[None, :] * L
              + jnp.arange(L, dtype=jnp.int32)[:, None]) * NPAD)
    src2 = src[None] + base2.reshape(L, B, 1, 1, 1, 1)
    zeros = jnp.zeros((RPT, D), jnp.float32)

    cur1, cur2 = _get_spmm2()(xall, src1, src2, dst, valr, zeros)
    c1 = cur1.reshape(B, L, NPAD, D)[:, :, :N]
    c2 = cur2.reshape(B, L, NPAD, D)[:, :, :N]

    acc, su, si = _stats(x0, c1, c2)
    mu, mi = _mid(uHyper, iHyper, su, si)

    ucw = u_concat_w.reshape(L, D, D)
    icw = i_concat_w.reshape(L, D, D)
    ue, ues = _emit_user(acc, mu, u_w, ucw)
    ie, ies = _emit_item(acc, mi, i_w, icw)
    return ue, ie, ues, ies
